# Initial kernel scaffold; baseline (speedup 1.0000x reference)
import jax
import jax.numpy as jnp
from jax.experimental import pallas as pl


def kernel(x, edge_index, weight, ln_gamma, ln_beta, W1):
    raise NotImplementedError("write your pallas kernel here")


# speedup vs baseline: 14.7055x; 14.7055x over previous
import functools

import jax
import jax.numpy as jnp
from jax import lax
from jax.experimental import pallas as pl
from jax.experimental.pallas import tpu as pltpu
from jax.experimental.pallas import tpu_sc as plsc

ALPHA = 0.1
N = 10000
E = 320000
H = 128

NC = 2
NS = 16
NW = NC * NS
CHUNK = 128
CHUNKS = 79
EPT = CHUNKS * CHUNK
E_PAD = EPT * NW
N_PAD = 10240
ROWS_PER_TILE = N_PAD // NS

_mesh = plsc.VectorSubcoreMesh(
    core_axis_name="c", subcore_axis_name="s", num_cores=NC, num_subcores=NS)


@functools.partial(
    pl.kernel,
    out_type=jax.ShapeDtypeStruct((NC, N_PAD), jnp.float32),
    mesh=_mesh,
    scratch_types=[
        pltpu.VMEM((CHUNKS, CHUNK), jnp.int32),
        pltpu.VMEM((CHUNKS, CHUNK), jnp.float32),
        pltpu.VMEM((ROWS_PER_TILE,), jnp.float32),
        pltpu.VMEM_SHARED((N_PAD,), jnp.float32),
    ],
)
def _sc_deg(col_hbm, w_hbm, out_hbm, col_v, w_v, zero_v, deg_sh):
    c = lax.axis_index("c")
    s = lax.axis_index("s")
    wid = s * NC + c
    pltpu.sync_copy(col_hbm.at[wid], col_v)
    pltpu.sync_copy(w_hbm.at[wid], w_v)

    zf = jnp.zeros((16,), jnp.float32)

    def zero_body(i, _):
        zero_v[pl.ds(i * 16, 16)] = zf
        return 0

    lax.fori_loop(0, ROWS_PER_TILE // 16, zero_body, 0)
    base = s * ROWS_PER_TILE
    pltpu.sync_copy(zero_v, deg_sh.at[pl.ds(base, ROWS_PER_TILE)])
    plsc.subcore_barrier()

    def chunk_body(j, _):
        pltpu.sync_copy(w_v.at[j], deg_sh.at[col_v.at[j]], add=True)
        return 0

    lax.fori_loop(0, CHUNKS, chunk_body, 0)
    plsc.subcore_barrier()
    pltpu.sync_copy(deg_sh.at[pl.ds(base, ROWS_PER_TILE)],
                    out_hbm.at[c, pl.ds(base, ROWS_PER_TILE)])


@functools.partial(
    pl.kernel,
    out_type=jax.ShapeDtypeStruct((NC, N_PAD, H), jnp.float32),
    mesh=_mesh,
    scratch_types=[
        pltpu.VMEM((CHUNKS, CHUNK), jnp.int32),
        pltpu.VMEM((CHUNKS, CHUNK), jnp.int32),
        pltpu.VMEM((CHUNKS, CHUNK), jnp.float32),
        pltpu.VMEM((CHUNK, H), jnp.float32),
        pltpu.SemaphoreType.DMA,
        pltpu.VMEM_SHARED((N_PAD, H), jnp.float32),
    ],
)
def _sc_msg(row_hbm, col_hbm, w_hbm, z_hbm, out_hbm,
            row_v, col_v, w_v, rows_v, sem, acc_sh):
    c = lax.axis_index("c")
    s = lax.axis_index("s")
    wid = s * NC + c
    pltpu.sync_copy(row_hbm.at[wid], row_v)
    pltpu.sync_copy(col_hbm.at[wid], col_v)
    pltpu.sync_copy(w_hbm.at[wid], w_v)

    zf = jnp.zeros((16,), jnp.float32)

    def zero_body(i, _):
        for k in range(H // 16):
            rows_v[i, pl.ds(k * 16, 16)] = zf
        return 0

    lax.fori_loop(0, CHUNK, zero_body, 0)
    base = s * ROWS_PER_TILE
    for q in range(ROWS_PER_TILE // CHUNK):
        pltpu.sync_copy(rows_v, acc_sh.at[pl.ds(base + q * CHUNK, CHUNK)])
    plsc.subcore_barrier()

    def chunk_body(j, _):
        pltpu.async_copy(z_hbm.at[row_v.at[j]], rows_v, sem).wait()

        def group_body(g, _):
            wrow = w_v[j, pl.ds(g * 16, 16)]
            for t in range(16):
                wv = jnp.full((16,), wrow[t], jnp.float32)
                e = g * 16 + t
                for k in range(H // 16):
                    sl = rows_v[e, pl.ds(k * 16, 16)]
                    rows_v[e, pl.ds(k * 16, 16)] = sl * wv
            return 0

        lax.fori_loop(0, CHUNK // 16, group_body, 0)
        pltpu.sync_copy(rows_v, acc_sh.at[col_v.at[j]], add=True)
        return 0

    lax.fori_loop(0, CHUNKS, chunk_body, 0)
    plsc.subcore_barrier()
    for q in range(ROWS_PER_TILE // CHUNK):
        pltpu.sync_copy(acc_sh.at[pl.ds(base + q * CHUNK, CHUNK)],
                        out_hbm.at[c, pl.ds(base + q * CHUNK, CHUNK)])


def _dinv_body(degp_ref, dinv_ref):
    d = degp_ref[0] + degp_ref[1] + 1.0
    dinv_ref[...] = lax.rsqrt(jnp.maximum(d, 1e-12))


def _ln_body(x_ref, g_ref, b_ref, dinv_ref, y_ref, z_ref):
    xb = x_ref[...]
    m = jnp.mean(xb, axis=1, keepdims=True)
    xc = xb - m
    v = jnp.mean(xc * xc, axis=1, keepdims=True)
    yb = xc * lax.rsqrt(v + 1e-5) * g_ref[...] + b_ref[...]
    y_ref[...] = yb
    z_ref[...] = yb * dinv_ref[...]


def _comb_body(s_ref, y_ref, dinv_ref, w1_ref, o_ref):
    sc = s_ref[0] + s_ref[1]
    di = dinv_ref[...]
    yb = y_ref[...]
    agg = di * sc + (di * di) * yb
    v = (1.0 - ALPHA) * agg + ALPHA * yb
    o_ref[...] = jnp.maximum(
        jnp.dot(v, w1_ref[...], preferred_element_type=jnp.float32,
                precision=lax.Precision.HIGHEST), 0.0)


_BLK = 2048
_GRID = N_PAD // _BLK


@jax.jit
def kernel(x, edge_index, weight, ln_gamma, ln_beta, W1):
    ei = edge_index.astype(jnp.int32)
    pad = E_PAD - E
    row3 = jnp.concatenate([ei[0], jnp.zeros((pad,), jnp.int32)]
                           ).reshape(NW, CHUNKS, CHUNK)
    col3 = jnp.concatenate([ei[1], jnp.zeros((pad,), jnp.int32)]
                           ).reshape(NW, CHUNKS, CHUNK)
    w3 = jnp.concatenate([weight, jnp.zeros((pad,), jnp.float32)]
                         ).reshape(NW, CHUNKS, CHUNK)
    x_pad = jnp.concatenate(
        [x, jnp.zeros((N_PAD - N, H), jnp.float32)], axis=0)

    degp = _sc_deg(col3, w3)

    dinv80 = pl.pallas_call(
        _dinv_body,
        out_shape=jax.ShapeDtypeStruct((N_PAD // 128, 128), jnp.float32),
    )(degp.reshape(NC, N_PAD // 128, 128))
    dinv_col = dinv80.reshape(N_PAD, 1)

    y_pad, z_pad = pl.pallas_call(
        _ln_body,
        grid=(_GRID,),
        in_specs=[
            pl.BlockSpec((_BLK, H), lambda i: (i, 0)),
            pl.BlockSpec((1, H), lambda i: (0, 0)),
            pl.BlockSpec((1, H), lambda i: (0, 0)),
            pl.BlockSpec((_BLK, 1), lambda i: (i, 0)),
        ],
        out_specs=[pl.BlockSpec((_BLK, H), lambda i: (i, 0))] * 2,
        out_shape=[jax.ShapeDtypeStruct((N_PAD, H), jnp.float32)] * 2,
    )(x_pad, ln_gamma.reshape(1, H), ln_beta.reshape(1, H), dinv_col)

    S = _sc_msg(row3, col3, w3, z_pad)

    out_pad = pl.pallas_call(
        _comb_body,
        grid=(_GRID,),
        in_specs=[
            pl.BlockSpec((NC, _BLK, H), lambda i: (0, i, 0)),
            pl.BlockSpec((_BLK, H), lambda i: (i, 0)),
            pl.BlockSpec((_BLK, 1), lambda i: (i, 0)),
            pl.BlockSpec((H, H), lambda i: (0, 0)),
        ],
        out_specs=pl.BlockSpec((_BLK, H), lambda i: (i, 0)),
        out_shape=jax.ShapeDtypeStruct((N_PAD, H), jnp.float32),
    )(S, y_pad, dinv_col, W1)

    return out_pad[:N]
